# baseline (speedup 1.0000x reference)
import functools

import jax
import jax.numpy as jnp
from jax.experimental import pallas as pl
from jax.experimental.pallas import tpu as pltpu

_B, _S, _D, _E, _K = 1, 2048, 1024, 8, 2
_N = _B * _S
_DFF = 4 * _D
_W_LOAD = 0.01

_TT = 256
_NT = _N // _TT
_DFFC = 512
_NKC = _DFF // _DFFC

_SQRT_2_OVER_PI = 0.7978845608028654
_INV_SQRT2 = 0.7071067811865476


def _gelu_tanh(x):
    return 0.5 * x * (1.0 + jnp.tanh(_SQRT_2_OVER_PI * (x + 0.044715 * x ** 3)))


def _softplus(x):
    return jnp.maximum(x, 0.0) + jnp.log(1.0 + jnp.exp(-jnp.abs(x)))


def _routing_body(x_ref, gw_ref, nw_ref, noise_ref,
                  wdense_ref, sel_ref, w12_ref, ll_ref, acc_ref):
    t = pl.program_id(0)
    x = x_ref[...]
    g = jnp.dot(x, gw_ref[...].T, preferred_element_type=jnp.float32)
    ns = _softplus(jnp.dot(x, nw_ref[...].T, preferred_element_type=jnp.float32))
    gl = g + noise_ref[...] * ns

    lanes = jax.lax.broadcasted_iota(jnp.int32, gl.shape, 1)
    m1 = jnp.max(gl, axis=1, keepdims=True)
    i1 = jnp.min(jnp.where(gl == m1, lanes, _E), axis=1, keepdims=True)
    glm = jnp.where(lanes == i1, -jnp.inf, gl)
    m2 = jnp.max(glm, axis=1, keepdims=True)
    i2 = jnp.min(jnp.where(glm == m2, lanes, _E), axis=1, keepdims=True)
    glm2 = jnp.where(lanes == i2, -jnp.inf, glm)
    m3 = jnp.max(glm2, axis=1, keepdims=True)

    e2 = jnp.exp(m2 - m1)
    w1 = 1.0 / (1.0 + e2)
    w2 = e2 / (1.0 + e2)
    wdense_ref[...] = (jnp.where(lanes == i1, w1, 0.0)
                       + jnp.where(lanes == i2, w2, 0.0))
    sel_ref[...] = jnp.where(lanes == 0, i1, jnp.where(lanes == 1, i2, 0))
    w12_ref[...] = jnp.where(lanes == 0, w1, jnp.where(lanes == 1, w2, 0.0))

    kth = jnp.where((lanes == i1) | (lanes == i2), m3, m2)
    z = (gl - kth) / jnp.maximum(ns, 1e-30)
    p = 0.5 * (1.0 + jax.lax.erf(z * _INV_SQRT2))

    @pl.when(t == 0)
    def _():
        acc_ref[...] = jnp.zeros_like(acc_ref)

    acc_ref[...] += jnp.sum(p, axis=0, keepdims=True)
    load = acc_ref[...]
    mean = jnp.mean(load)
    var = jnp.sum((load - mean) ** 2) / (_E - 1)
    ll_ref[...] = jnp.full((1, 1), _W_LOAD * var / (mean * mean), jnp.float32)


def _routing(xs, gate_w, noise_w, noise):
    return pl.pallas_call(
        _routing_body,
        grid=(_NT,),
        in_specs=[
            pl.BlockSpec((_TT, _D), lambda t: (t, 0)),
            pl.BlockSpec((_E, _D), lambda t: (0, 0)),
            pl.BlockSpec((_E, _D), lambda t: (0, 0)),
            pl.BlockSpec((_TT, _E), lambda t: (t, 0)),
        ],
        out_specs=[
            pl.BlockSpec((_TT, _E), lambda t: (t, 0)),
            pl.BlockSpec((_TT, _E), lambda t: (t, 0)),
            pl.BlockSpec((_TT, _E), lambda t: (t, 0)),
            pl.BlockSpec((1, 1), lambda t: (0, 0)),
        ],
        out_shape=[
            jax.ShapeDtypeStruct((_N, _E), jnp.float32),
            jax.ShapeDtypeStruct((_N, _E), jnp.int32),
            jax.ShapeDtypeStruct((_N, _E), jnp.float32),
            jax.ShapeDtypeStruct((1, 1), jnp.float32),
        ],
        scratch_shapes=[pltpu.VMEM((1, _E), jnp.float32)],
    )(xs, gate_w, noise_w, noise)


def _ffn_body(wd_ref, x_ref, wfc_ref, bfc_ref, wproj_ref, bproj_ref, out_ref):
    e = pl.program_id(1)
    kc = pl.program_id(2)
    x = x_ref[...]
    h = jnp.dot(x, wfc_ref[0].T, preferred_element_type=jnp.float32) + bfc_ref[0]
    h = _gelu_tanh(h)
    part = jnp.dot(h, wproj_ref[0].T, preferred_element_type=jnp.float32)

    lanes = jax.lax.broadcasted_iota(jnp.int32, wd_ref.shape, 1)
    wcol = jnp.sum(jnp.where(lanes == e, wd_ref[...], 0.0), axis=1, keepdims=True)

    part = jnp.where(kc == 0, part + bproj_ref[0], part)
    contrib = part * wcol

    @pl.when((e == 0) & (kc == 0))
    def _():
        out_ref[...] = contrib

    @pl.when((e > 0) | (kc > 0))
    def _():
        out_ref[...] += contrib


def _ffn(wdense, xs, wfc, bfc, wproj, bproj):
    return pl.pallas_call(
        _ffn_body,
        grid=(_NT, _E, _NKC),
        in_specs=[
            pl.BlockSpec((_TT, _E), lambda t, e, kc: (t, 0)),
            pl.BlockSpec((_TT, _D), lambda t, e, kc: (t, 0)),
            pl.BlockSpec((1, _DFFC, _D), lambda t, e, kc: (e, kc, 0)),
            pl.BlockSpec((1, 1, _DFFC), lambda t, e, kc: (e * _NKC + kc, 0, 0)),
            pl.BlockSpec((1, _D, _DFFC), lambda t, e, kc: (e, 0, kc)),
            pl.BlockSpec((1, 1, _D), lambda t, e, kc: (e, 0, 0)),
        ],
        out_specs=pl.BlockSpec((_TT, _D), lambda t, e, kc: (t, 0)),
        out_shape=jax.ShapeDtypeStruct((_N, _D), jnp.float32),
        compiler_params=pltpu.CompilerParams(
            dimension_semantics=("arbitrary", "arbitrary", "arbitrary")),
    )(wdense, xs, wfc,
      bfc.reshape(_E * _NKC, 1, _DFFC),
      wproj,
      bproj.reshape(_E, 1, _D))


_TM = 256
_NR = _N * _K
_NT2 = _NR // _TM
_NSLOT = _NT2 + _E - 1
_DFFC2 = 512
_NKC2 = _DFF // _DFFC2


def _gffn_body(off_ref,
               re_ref, rw_ref, xg_ref, wfc_ref, bfc_ref, wproj_ref, bproj_ref,
               out_ref):
    kc = pl.program_id(0)
    e = pl.program_id(1)

    @pl.when((kc == 0) & (e == 0))
    def _():
        out_ref[...] = jnp.zeros_like(out_ref)

    start = off_ref[e]
    end = off_ref[e + 1]
    c0 = start // _TM
    c1 = (end + _TM - 1) // _TM

    def chunk(i, carry):
        base = (c0 + i) * _TM
        xb = xg_ref[pl.ds(base, _TM), :]
        h = jnp.dot(xb, wfc_ref[0].T, preferred_element_type=jnp.float32) + bfc_ref[0]
        h = _gelu_tanh(h).astype(jnp.bfloat16)
        part = jnp.dot(h, wproj_ref[0].T, preferred_element_type=jnp.float32)
        mask = re_ref[pl.ds(base, _TM), :] == e
        wrow = jnp.where(mask, rw_ref[pl.ds(base, _TM), :], 0.0)
        part = jnp.where(kc == 0, part + bproj_ref[0], part)
        out_ref[pl.ds(base, _TM), :] += part * wrow
        return carry

    jax.lax.fori_loop(0, c1 - c0, chunk, 0)


def _gffn(offsets, row_e, row_w, xg, wfc, bfc, wproj, bproj):
    grid_spec = pltpu.PrefetchScalarGridSpec(
        num_scalar_prefetch=1,
        grid=(_NKC2, _E),
        in_specs=[
            pl.BlockSpec((_NR, 1), lambda kc, e, off: (0, 0)),
            pl.BlockSpec((_NR, 1), lambda kc, e, off: (0, 0)),
            pl.BlockSpec((_NR, _D), lambda kc, e, off: (0, 0)),
            pl.BlockSpec((1, _DFFC2, _D), lambda kc, e, off: (e, kc, 0)),
            pl.BlockSpec((1, 1, _DFFC2), lambda kc, e, off: (e * _NKC2 + kc, 0, 0)),
            pl.BlockSpec((1, _D, _DFFC2), lambda kc, e, off: (e, 0, kc)),
            pl.BlockSpec((1, 1, _D), lambda kc, e, off: (e, 0, 0)),
        ],
        out_specs=pl.BlockSpec((_NR, _D), lambda kc, e, off: (0, 0)),
    )
    return pl.pallas_call(
        _gffn_body,
        grid_spec=grid_spec,
        out_shape=jax.ShapeDtypeStruct((_NR, _D), jnp.float32),
        compiler_params=pltpu.CompilerParams(
            dimension_semantics=("arbitrary", "arbitrary")),
    )(offsets,
      row_e.reshape(_NR, 1), row_w.reshape(_NR, 1), xg,
      wfc.astype(jnp.bfloat16),
      bfc.reshape(_E * _NKC2, 1, _DFFC2),
      wproj.astype(jnp.bfloat16),
      bproj.reshape(_E, 1, _D))


def _dispatch_plan(sel, w12):
    e_flat = jnp.concatenate([sel[:, 0], sel[:, 1]]).astype(jnp.int32)
    w_flat = jnp.concatenate([w12[:, 0], w12[:, 1]])
    perm = jnp.argsort(e_flat)
    row_e = e_flat[perm]
    row_w = w_flat[perm]
    row_tok = (perm % _N).astype(jnp.int32)
    pos = jnp.zeros((_NR,), jnp.int32).at[perm].set(
        jnp.arange(_NR, dtype=jnp.int32))
    counts = jnp.sum((e_flat[:, None] ==
                      jnp.arange(_E, dtype=jnp.int32)[None, :]).astype(jnp.int32),
                     axis=0)
    offsets = jnp.concatenate([jnp.zeros((1,), jnp.int32),
                               jnp.cumsum(counts).astype(jnp.int32)])
    return offsets, row_e, row_w, row_tok, pos


def kernel(x, noise, gate_w, noise_w, wfc, bfc, wproj, bproj):
    xs = x.reshape(-1, x.shape[-1])
    wdense, sel, w12, ll = _routing(xs, gate_w, noise_w, noise)
    offsets, row_e, row_w, row_tok, pos = _dispatch_plan(sel[:, :2], w12[:, :2])
    xg = jnp.take(xs.astype(jnp.bfloat16), row_tok, axis=0)
    yg = _gffn(offsets, row_e, row_w, xg, wfc, bfc, wproj, bproj)
    out = yg[pos[:_N]] + yg[pos[_N:]]
    return out.reshape(x.shape), ll.reshape(())


# speedup vs baseline: 1.0013x; 1.0013x over previous
import functools

import jax
import jax.numpy as jnp
from jax.experimental import pallas as pl
from jax.experimental.pallas import tpu as pltpu

_B, _S, _D, _E, _K = 1, 2048, 1024, 8, 2
_N = _B * _S
_DFF = 4 * _D
_W_LOAD = 0.01

_TT = 256
_NT = _N // _TT
_DFFC = 512
_NKC = _DFF // _DFFC

_SQRT_2_OVER_PI = 0.7978845608028654
_INV_SQRT2 = 0.7071067811865476


def _gelu_tanh(x):
    return 0.5 * x * (1.0 + jnp.tanh(_SQRT_2_OVER_PI * (x + 0.044715 * x ** 3)))


def _softplus(x):
    return jnp.maximum(x, 0.0) + jnp.log(1.0 + jnp.exp(-jnp.abs(x)))


def _routing_body(x_ref, gw_ref, nw_ref, noise_ref,
                  wdense_ref, sel_ref, w12_ref, ll_ref, acc_ref):
    t = pl.program_id(0)
    x = x_ref[...]
    g = jnp.dot(x, gw_ref[...].T, preferred_element_type=jnp.float32)
    ns = _softplus(jnp.dot(x, nw_ref[...].T, preferred_element_type=jnp.float32))
    gl = g + noise_ref[...] * ns

    lanes = jax.lax.broadcasted_iota(jnp.int32, gl.shape, 1)
    m1 = jnp.max(gl, axis=1, keepdims=True)
    i1 = jnp.min(jnp.where(gl == m1, lanes, _E), axis=1, keepdims=True)
    glm = jnp.where(lanes == i1, -jnp.inf, gl)
    m2 = jnp.max(glm, axis=1, keepdims=True)
    i2 = jnp.min(jnp.where(glm == m2, lanes, _E), axis=1, keepdims=True)
    glm2 = jnp.where(lanes == i2, -jnp.inf, glm)
    m3 = jnp.max(glm2, axis=1, keepdims=True)

    e2 = jnp.exp(m2 - m1)
    w1 = 1.0 / (1.0 + e2)
    w2 = e2 / (1.0 + e2)
    wdense_ref[...] = (jnp.where(lanes == i1, w1, 0.0)
                       + jnp.where(lanes == i2, w2, 0.0))
    sel_ref[...] = jnp.where(lanes == 0, i1, jnp.where(lanes == 1, i2, 0))
    w12_ref[...] = jnp.where(lanes == 0, w1, jnp.where(lanes == 1, w2, 0.0))

    kth = jnp.where((lanes == i1) | (lanes == i2), m3, m2)
    z = (gl - kth) / jnp.maximum(ns, 1e-30)
    p = 0.5 * (1.0 + jax.lax.erf(z * _INV_SQRT2))

    @pl.when(t == 0)
    def _():
        acc_ref[...] = jnp.zeros_like(acc_ref)

    acc_ref[...] += jnp.sum(p, axis=0, keepdims=True)
    load = acc_ref[...]
    mean = jnp.mean(load)
    var = jnp.sum((load - mean) ** 2) / (_E - 1)
    ll_ref[...] = jnp.full((1, 1), _W_LOAD * var / (mean * mean), jnp.float32)


def _routing(xs, gate_w, noise_w, noise):
    return pl.pallas_call(
        _routing_body,
        grid=(_NT,),
        in_specs=[
            pl.BlockSpec((_TT, _D), lambda t: (t, 0)),
            pl.BlockSpec((_E, _D), lambda t: (0, 0)),
            pl.BlockSpec((_E, _D), lambda t: (0, 0)),
            pl.BlockSpec((_TT, _E), lambda t: (t, 0)),
        ],
        out_specs=[
            pl.BlockSpec((_TT, _E), lambda t: (t, 0)),
            pl.BlockSpec((_TT, _E), lambda t: (t, 0)),
            pl.BlockSpec((_TT, _E), lambda t: (t, 0)),
            pl.BlockSpec((1, 1), lambda t: (0, 0)),
        ],
        out_shape=[
            jax.ShapeDtypeStruct((_N, _E), jnp.float32),
            jax.ShapeDtypeStruct((_N, _E), jnp.int32),
            jax.ShapeDtypeStruct((_N, _E), jnp.float32),
            jax.ShapeDtypeStruct((1, 1), jnp.float32),
        ],
        scratch_shapes=[pltpu.VMEM((1, _E), jnp.float32)],
    )(xs, gate_w, noise_w, noise)


def _ffn_body(wd_ref, x_ref, wfc_ref, bfc_ref, wproj_ref, bproj_ref, out_ref):
    e = pl.program_id(1)
    kc = pl.program_id(2)
    x = x_ref[...]
    h = jnp.dot(x, wfc_ref[0].T, preferred_element_type=jnp.float32) + bfc_ref[0]
    h = _gelu_tanh(h)
    part = jnp.dot(h, wproj_ref[0].T, preferred_element_type=jnp.float32)

    lanes = jax.lax.broadcasted_iota(jnp.int32, wd_ref.shape, 1)
    wcol = jnp.sum(jnp.where(lanes == e, wd_ref[...], 0.0), axis=1, keepdims=True)

    part = jnp.where(kc == 0, part + bproj_ref[0], part)
    contrib = part * wcol

    @pl.when((e == 0) & (kc == 0))
    def _():
        out_ref[...] = contrib

    @pl.when((e > 0) | (kc > 0))
    def _():
        out_ref[...] += contrib


def _ffn(wdense, xs, wfc, bfc, wproj, bproj):
    return pl.pallas_call(
        _ffn_body,
        grid=(_NT, _E, _NKC),
        in_specs=[
            pl.BlockSpec((_TT, _E), lambda t, e, kc: (t, 0)),
            pl.BlockSpec((_TT, _D), lambda t, e, kc: (t, 0)),
            pl.BlockSpec((1, _DFFC, _D), lambda t, e, kc: (e, kc, 0)),
            pl.BlockSpec((1, 1, _DFFC), lambda t, e, kc: (e * _NKC + kc, 0, 0)),
            pl.BlockSpec((1, _D, _DFFC), lambda t, e, kc: (e, 0, kc)),
            pl.BlockSpec((1, 1, _D), lambda t, e, kc: (e, 0, 0)),
        ],
        out_specs=pl.BlockSpec((_TT, _D), lambda t, e, kc: (t, 0)),
        out_shape=jax.ShapeDtypeStruct((_N, _D), jnp.float32),
        compiler_params=pltpu.CompilerParams(
            dimension_semantics=("arbitrary", "arbitrary", "arbitrary")),
    )(wdense, xs, wfc,
      bfc.reshape(_E * _NKC, 1, _DFFC),
      wproj,
      bproj.reshape(_E, 1, _D))


_TM = 256
_NR = _N * _K
_NT2 = _NR // _TM
_NSLOT = _NT2 + _E - 1
_DFFC2 = 2048
_NKC2 = _DFF // _DFFC2


def _gffn_body(off_ref,
               re_ref, rw_ref, xg_ref, wfc_ref, bfc_ref, wproj_ref, bproj_ref,
               out_ref):
    kc = pl.program_id(0)
    e = pl.program_id(1)

    @pl.when((kc == 0) & (e == 0))
    def _():
        out_ref[...] = jnp.zeros_like(out_ref)

    start = off_ref[e]
    end = off_ref[e + 1]
    c0 = start // _TM
    c1 = (end + _TM - 1) // _TM

    def chunk(i, carry):
        base = (c0 + i) * _TM
        xb = xg_ref[pl.ds(base, _TM), :]
        h = jnp.dot(xb, wfc_ref[0], preferred_element_type=jnp.float32) + bfc_ref[0]
        h = _gelu_tanh(h).astype(jnp.bfloat16)
        part = jnp.dot(h, wproj_ref[0], preferred_element_type=jnp.float32)
        mask = re_ref[pl.ds(base, _TM), :] == e
        wrow = jnp.where(mask, rw_ref[pl.ds(base, _TM), :], 0.0)
        part = jnp.where(kc == 0, part + bproj_ref[0], part)
        out_ref[pl.ds(base, _TM), :] += part * wrow
        return carry

    jax.lax.fori_loop(0, c1 - c0, chunk, 0)


def _gffn(offsets, row_e, row_w, xg, wfc, bfc, wproj, bproj):
    grid_spec = pltpu.PrefetchScalarGridSpec(
        num_scalar_prefetch=1,
        grid=(_NKC2, _E),
        in_specs=[
            pl.BlockSpec((_NR, 1), lambda kc, e, off: (0, 0)),
            pl.BlockSpec((_NR, 1), lambda kc, e, off: (0, 0)),
            pl.BlockSpec((_NR, _D), lambda kc, e, off: (0, 0)),
            pl.BlockSpec((1, _D, _DFFC2), lambda kc, e, off: (e, 0, kc)),
            pl.BlockSpec((1, 1, _DFFC2), lambda kc, e, off: (e * _NKC2 + kc, 0, 0)),
            pl.BlockSpec((1, _DFFC2, _D), lambda kc, e, off: (e, kc, 0)),
            pl.BlockSpec((1, 1, _D), lambda kc, e, off: (e, 0, 0)),
        ],
        out_specs=pl.BlockSpec((_NR, _D), lambda kc, e, off: (0, 0)),
    )
    return pl.pallas_call(
        _gffn_body,
        grid_spec=grid_spec,
        out_shape=jax.ShapeDtypeStruct((_NR, _D), jnp.float32),
        compiler_params=pltpu.CompilerParams(
            dimension_semantics=("arbitrary", "arbitrary")),
    )(offsets,
      row_e.reshape(_NR, 1), row_w.reshape(_NR, 1), xg,
      wfc.swapaxes(1, 2).astype(jnp.bfloat16),
      bfc.reshape(_E * _NKC2, 1, _DFFC2),
      wproj.swapaxes(1, 2).astype(jnp.bfloat16),
      bproj.reshape(_E, 1, _D))


def _dispatch_plan(sel, w12):
    e_flat = jnp.concatenate([sel[:, 0], sel[:, 1]]).astype(jnp.int32)
    w_flat = jnp.concatenate([w12[:, 0], w12[:, 1]])
    perm = jnp.argsort(e_flat)
    row_e = e_flat[perm]
    row_w = w_flat[perm]
    row_tok = (perm % _N).astype(jnp.int32)
    pos = jnp.zeros((_NR,), jnp.int32).at[perm].set(
        jnp.arange(_NR, dtype=jnp.int32))
    counts = jnp.sum((e_flat[:, None] ==
                      jnp.arange(_E, dtype=jnp.int32)[None, :]).astype(jnp.int32),
                     axis=0)
    offsets = jnp.concatenate([jnp.zeros((1,), jnp.int32),
                               jnp.cumsum(counts).astype(jnp.int32)])
    return offsets, row_e, row_w, row_tok, pos


def kernel(x, noise, gate_w, noise_w, wfc, bfc, wproj, bproj):
    xs = x.reshape(-1, x.shape[-1])
    wdense, sel, w12, ll = _routing(xs, gate_w, noise_w, noise)
    offsets, row_e, row_w, row_tok, pos = _dispatch_plan(sel[:, :2], w12[:, :2])
    xg = jnp.take(xs.astype(jnp.bfloat16), row_tok, axis=0)
    yg = _gffn(offsets, row_e, row_w, xg, wfc, bfc, wproj, bproj)
    out = yg[pos[:_N]] + yg[pos[_N:]]
    return out.reshape(x.shape), ll.reshape(())


# speedup vs baseline: 1.3139x; 1.3121x over previous
import functools

import jax
import jax.numpy as jnp
from jax import lax
from jax.experimental import pallas as pl
from jax.experimental.pallas import tpu as pltpu
from jax.experimental.pallas import tpu_sc as plsc

_B, _S, _D, _E, _K = 1, 2048, 1024, 8, 2
_N = _B * _S
_DFF = 4 * _D
_W_LOAD = 0.01

_TT = 256
_NT = _N // _TT

_SQRT_2_OVER_PI = 0.7978845608028654
_INV_SQRT2 = 0.7071067811865476


def _gelu_tanh(x):
    return 0.5 * x * (1.0 + jnp.tanh(_SQRT_2_OVER_PI * (x + 0.044715 * x ** 3)))


def _softplus(x):
    return jnp.maximum(x, 0.0) + jnp.log(1.0 + jnp.exp(-jnp.abs(x)))


def _routing_body(x_ref, gw_ref, nw_ref, noise_ref,
                  sel_ref, w12_ref, ll_ref, acc_ref):
    t = pl.program_id(0)
    x = x_ref[...]
    g = jnp.dot(x, gw_ref[...].T, preferred_element_type=jnp.float32)
    ns = _softplus(jnp.dot(x, nw_ref[...].T, preferred_element_type=jnp.float32))
    gl = g + noise_ref[...] * ns

    lanes = jax.lax.broadcasted_iota(jnp.int32, gl.shape, 1)
    m1 = jnp.max(gl, axis=1, keepdims=True)
    i1 = jnp.min(jnp.where(gl == m1, lanes, _E), axis=1, keepdims=True)
    glm = jnp.where(lanes == i1, -jnp.inf, gl)
    m2 = jnp.max(glm, axis=1, keepdims=True)
    i2 = jnp.min(jnp.where(glm == m2, lanes, _E), axis=1, keepdims=True)
    glm2 = jnp.where(lanes == i2, -jnp.inf, glm)
    m3 = jnp.max(glm2, axis=1, keepdims=True)

    e2 = jnp.exp(m2 - m1)
    w1 = 1.0 / (1.0 + e2)
    w2 = e2 / (1.0 + e2)
    sel_ref[...] = jnp.where(lanes == 0, i1, jnp.where(lanes == 1, i2, 0))
    w12_ref[...] = jnp.where(lanes == 0, w1, jnp.where(lanes == 1, w2, 0.0))

    kth = jnp.where((lanes == i1) | (lanes == i2), m3, m2)
    z = (gl - kth) / jnp.maximum(ns, 1e-30)
    p = 0.5 * (1.0 + jax.lax.erf(z * _INV_SQRT2))

    @pl.when(t == 0)
    def _():
        acc_ref[...] = jnp.zeros_like(acc_ref)

    acc_ref[...] += jnp.sum(p, axis=0, keepdims=True)
    load = acc_ref[...]
    mean = jnp.mean(load)
    var = jnp.sum((load - mean) ** 2) / (_E - 1)
    ll_ref[...] = jnp.full((1, 1), _W_LOAD * var / (mean * mean), jnp.float32)


def _routing(xs, gate_w, noise_w, noise):
    return pl.pallas_call(
        _routing_body,
        grid=(_NT,),
        in_specs=[
            pl.BlockSpec((_TT, _D), lambda t: (t, 0)),
            pl.BlockSpec((_E, _D), lambda t: (0, 0)),
            pl.BlockSpec((_E, _D), lambda t: (0, 0)),
            pl.BlockSpec((_TT, _E), lambda t: (t, 0)),
        ],
        out_specs=[
            pl.BlockSpec((_TT, _E), lambda t: (t, 0)),
            pl.BlockSpec((_TT, _E), lambda t: (t, 0)),
            pl.BlockSpec((1, 1), lambda t: (0, 0)),
        ],
        out_shape=[
            jax.ShapeDtypeStruct((_N, _E), jnp.int32),
            jax.ShapeDtypeStruct((_N, _E), jnp.float32),
            jax.ShapeDtypeStruct((1, 1), jnp.float32),
        ],
        scratch_shapes=[pltpu.VMEM((1, _E), jnp.float32)],
    )(xs, gate_w, noise_w, noise)


_NR = _N * _K
_NW = 32
_EPW = _NR // _NW
_VPW = _EPW // 16


def _hist_body(ef2_hbm, histp_hbm, myev_v, hp_v):
    c = lax.axis_index("c")
    s = lax.axis_index("s")
    wid = s * 2 + c
    pltpu.sync_copy(ef2_hbm.at[wid], myev_v)
    for b in range(_E):
        hv = jnp.zeros((16,), jnp.int32)
        for j in range(_VPW):
            v = myev_v[pl.ds(j * 16, 16)]
            hv = hv + 1 - jnp.minimum(jnp.abs(v - b), 1)
        hp_v[pl.ds(b * 16, 16)] = hv
    pltpu.sync_copy(hp_v, histp_hbm.at[wid])


def _rank_body(ef2_hbm, base_hbm, xs_hbm, xg_hbm, pos2_hbm,
               myev_v, bv_v, buf_v, pos_a, pos_b):
    c = lax.axis_index("c")
    s = lax.axis_index("s")
    wid = s * 2 + c
    pltpu.sync_copy(ef2_hbm.at[wid], myev_v)
    pltpu.sync_copy(base_hbm.at[wid], bv_v)
    bv = bv_v[...]

    lane = lax.iota(jnp.int32, 16)

    def _psum(csv):
        for k in (1, 2, 4, 8):
            idxk = jnp.maximum(lane - k, 0)
            zk = jnp.minimum(jnp.maximum(lane - k + 1, 0), 1)
            csv = csv + csv.at[idxk].get(mode="promise_in_bounds") * zk
        return csv

    rc = []
    for b in range(_E):
        idx_b = jnp.zeros((16,), jnp.int32) + b
        rc.append(bv.at[idx_b].get(mode="promise_in_bounds"))
    last = jnp.zeros((16,), jnp.int32) + 15
    for j in range(_VPW):
        v = myev_v[pl.ds(j * 16, 16)]
        posv = jnp.zeros((16,), jnp.int32)
        for b in range(_E):
            mi = 1 - jnp.minimum(jnp.abs(v - b), 1)
            cs = _psum(mi)
            posv = posv + mi * (rc[b] + cs - 1)
            rc[b] = rc[b] + cs.at[last].get(mode="promise_in_bounds")
        half = pos_a if j < _VPW // 2 else pos_b
        half[pl.ds((j % (_VPW // 2)) * 16, 16)] = posv

    half_rows = _EPW // 2
    pltpu.sync_copy(pos_a, pos2_hbm.at[2 * wid])
    pltpu.sync_copy(pos_b, pos2_hbm.at[2 * wid + 1])

    tok0 = jnp.where(wid >= _NW // 2, wid * _EPW - _N, wid * _EPW)
    pltpu.sync_copy(xs_hbm.at[pl.ds(tok0, half_rows)], buf_v)
    pltpu.sync_copy(buf_v, xg_hbm.at[pos_a])
    pltpu.sync_copy(xs_hbm.at[pl.ds(tok0 + half_rows, half_rows)], buf_v)
    pltpu.sync_copy(buf_v, xg_hbm.at[pos_b])


def _dispatch(e_flat, xs):
    mesh = plsc.VectorSubcoreMesh(core_axis_name="c", subcore_axis_name="s")
    hist_k = functools.partial(
        pl.kernel, mesh=mesh,
        out_type=[jax.ShapeDtypeStruct((_NW, _E * 16), jnp.int32)],
        scratch_types=[
            pltpu.VMEM((_EPW,), jnp.int32),
            pltpu.VMEM((_E * 16,), jnp.int32),
        ],
    )(_hist_body)
    (histp,) = hist_k(e_flat.reshape(_NW, _EPW))

    hist = histp.reshape(_NW, _E, 16).sum(axis=2)
    counts = hist.sum(axis=0)
    bin_base = jnp.concatenate([jnp.zeros((1,), jnp.int32),
                                jnp.cumsum(counts).astype(jnp.int32)])
    wprefix = jnp.cumsum(hist, axis=0) - hist
    bases = bin_base[None, :_E] + wprefix.astype(jnp.int32)
    bases16 = jnp.concatenate(
        [bases, jnp.zeros((_NW, 16 - _E), jnp.int32)], axis=1)

    rank_k = functools.partial(
        pl.kernel, mesh=mesh,
        out_type=[
            jax.ShapeDtypeStruct((_NR, _D), jnp.float32),
            jax.ShapeDtypeStruct((2 * _NW, _EPW // 2), jnp.int32),
        ],
        scratch_types=[
            pltpu.VMEM((_EPW,), jnp.int32),
            pltpu.VMEM((16,), jnp.int32),
            pltpu.VMEM((_EPW // 2, _D), jnp.float32),
            pltpu.VMEM((_EPW // 2,), jnp.int32),
            pltpu.VMEM((_EPW // 2,), jnp.int32),
        ],
    )(_rank_body)
    xg, pos2 = rank_k(e_flat.reshape(_NW, _EPW), bases16, xs)
    return xg, pos2.reshape(_NR), bin_base


_TM = 256
_DFFC2 = 2048
_NKC2 = _DFF // _DFFC2


def _gffn_body(off_ref,
               xg_ref, wfc_ref, bfc_ref, wproj_ref, bproj_ref,
               out_ref):
    kc = pl.program_id(0)
    e = pl.program_id(1)

    @pl.when((kc == 0) & (e == 0))
    def _():
        out_ref[...] = jnp.zeros_like(out_ref)

    start = off_ref[e]
    end = off_ref[e + 1]
    c0 = start // _TM
    c1 = (end + _TM - 1) // _TM

    def chunk(i, carry):
        base = (c0 + i) * _TM
        xb = xg_ref[pl.ds(base, _TM), :].astype(jnp.bfloat16)
        h = jnp.dot(xb, wfc_ref[0].T, preferred_element_type=jnp.float32) + bfc_ref[0]
        h = _gelu_tanh(h).astype(jnp.bfloat16)
        part = jnp.dot(h, wproj_ref[0].T, preferred_element_type=jnp.float32)
        rows = base + jax.lax.broadcasted_iota(jnp.int32, (_TM, 1), 0)
        inmask = ((rows >= start) & (rows < end)).astype(jnp.float32)
        part = jnp.where(kc == 0, part + bproj_ref[0], part)
        out_ref[pl.ds(base, _TM), :] += part * inmask
        return carry

    jax.lax.fori_loop(0, c1 - c0, chunk, 0)


def _gffn(offsets, xg, wfc, bfc, wproj, bproj):
    grid_spec = pltpu.PrefetchScalarGridSpec(
        num_scalar_prefetch=1,
        grid=(_NKC2, _E),
        in_specs=[
            pl.BlockSpec((_NR, _D), lambda kc, e, off: (0, 0)),
            pl.BlockSpec((1, _DFFC2, _D), lambda kc, e, off: (e, kc, 0)),
            pl.BlockSpec((1, 1, _DFFC2), lambda kc, e, off: (e * _NKC2 + kc, 0, 0)),
            pl.BlockSpec((1, _D, _DFFC2), lambda kc, e, off: (e, 0, kc)),
            pl.BlockSpec((1, 1, _D), lambda kc, e, off: (e, 0, 0)),
        ],
        out_specs=pl.BlockSpec((_NR, _D), lambda kc, e, off: (0, 0)),
    )
    return pl.pallas_call(
        _gffn_body,
        grid_spec=grid_spec,
        out_shape=jax.ShapeDtypeStruct((_NR, _D), jnp.float32),
        compiler_params=pltpu.CompilerParams(
            dimension_semantics=("arbitrary", "arbitrary")),
    )(offsets, xg,
      wfc.astype(jnp.bfloat16),
      bfc.reshape(_E * _NKC2, 1, _DFFC2),
      wproj.astype(jnp.bfloat16),
      bproj.reshape(_E, 1, _D))


def kernel(x, noise, gate_w, noise_w, wfc, bfc, wproj, bproj):
    xs = x.reshape(-1, x.shape[-1])
    sel, w12, ll = _routing(xs, gate_w, noise_w, noise)
    e_flat = jnp.concatenate([sel[:, 0], sel[:, 1]]).astype(jnp.int32)
    w_flat = jnp.concatenate([w12[:, 0], w12[:, 1]])
    xg, pos, offsets = _dispatch(e_flat, xs)
    yg = _gffn(offsets, xg, wfc, bfc, wproj, bproj)
    out = (w_flat[:_N, None] * yg[pos[:_N]]
           + w_flat[_N:, None] * yg[pos[_N:]])
    return out.reshape(x.shape), ll.reshape(())
